# Initial kernel scaffold; baseline (speedup 1.0000x reference)
import jax
import jax.numpy as jnp
from jax.experimental import pallas as pl


def kernel(x_send, x_rec, index_send, index_rec, edge_attr, W1, b1, bn_gamma, bn_beta, We, be):
    raise NotImplementedError("write your pallas kernel here")


# speedup vs baseline: 2.6316x; 2.6316x over previous
import functools

import jax
import jax.numpy as jnp
from jax import lax
from jax.experimental import pallas as pl
from jax.experimental.pallas import tpu as pltpu
from jax.experimental.pallas import tpu_sc as plsc

_N = 10000
_E = 320000
_H = 128
_I = 16

_NC = 2
_NS = 16
_NW = _NC * _NS
_EPT = _E // _NW
_CHUNK = 80
_NCHUNK = _EPT // _CHUNK
_CB = _H // 16
_NP = 10112
_RPT = _NP // _NS

_mesh = plsc.VectorSubcoreMesh(
    core_axis_name="c", subcore_axis_name="s", num_cores=_NC, num_subcores=_NS
)


def _pq_body(xs_ref, xr_ref, wa_ref, wb_ref, p_ref, q_ref):
    p_ref[...] = jnp.dot(xs_ref[...], wa_ref[...], preferred_element_type=jnp.float32)
    q_ref[...] = jnp.dot(xr_ref[...], wb_ref[...], preferred_element_type=jnp.float32)


_RBLK = 8000


def _r_body(ea_ref, wc_ref, b1_ref, r_ref):
    r_ref[...] = (
        jnp.dot(ea_ref[...], wc_ref[...], preferred_element_type=jnp.float32)
        + b1_ref[...]
    )


def _add_body(a_ref, b_ref, o_ref):
    o_ref[...] = a_ref[...] + b_ref[...]


@functools.partial(
    pl.kernel,
    out_type=(
        jax.ShapeDtypeStruct((_E, _H), jnp.float32),
        jax.ShapeDtypeStruct((_NW, 2, _H), jnp.float32),
    ),
    mesh=_mesh,
    scratch_types=[
        pltpu.VMEM((_CHUNK,), jnp.int32),
        pltpu.VMEM((_CHUNK,), jnp.int32),
        pltpu.VMEM((_CHUNK, _H), jnp.float32),
        pltpu.VMEM((_CHUNK, _H), jnp.float32),
        pltpu.VMEM((_CHUNK, _H), jnp.float32),
        pltpu.VMEM((_CHUNK, _H), jnp.float32),
        pltpu.VMEM((2, _H), jnp.float32),
        pltpu.SemaphoreType.DMA,
        pltpu.SemaphoreType.DMA,
        pltpu.SemaphoreType.DMA,
    ],
)
def _sc_phase1(p_hbm, q_hbm, rb_hbm, isend_hbm, irec_hbm, h_hbm, stats_hbm,
               idx_v, idx2_v, bufp, bufq, bufr, bufh, statv, sem_p, sem_q, sem_r):
    wid = lax.axis_index("s") * _NC + lax.axis_index("c")
    base = wid * _EPT

    def chunk_body(it, accs):
        off = base + it * _CHUNK
        pltpu.sync_copy(isend_hbm.at[pl.ds(off, _CHUNK)], idx_v)
        pltpu.sync_copy(irec_hbm.at[pl.ds(off, _CHUNK)], idx2_v)
        cp = pltpu.async_copy(p_hbm.at[idx_v], bufp, sem_p)
        cq = pltpu.async_copy(q_hbm.at[idx2_v], bufq, sem_q)
        cr = pltpu.async_copy(rb_hbm.at[pl.ds(off, _CHUNK)], bufr, sem_r)
        cp.wait()
        cq.wait()
        cr.wait()

        def edge_body(e, a):
            out = list(a)
            for cb in range(_CB):
                sl = pl.ds(cb * 16, 16)
                t = bufp[e, sl] + bufq[e, sl] + bufr[e, sl]
                bufh[e, sl] = t
                out[cb] = out[cb] + t
                out[_CB + cb] = out[_CB + cb] + t * t
            return tuple(out)

        accs = lax.fori_loop(0, _CHUNK, edge_body, accs)
        pltpu.sync_copy(bufh, h_hbm.at[pl.ds(off, _CHUNK)])
        return accs

    accs = tuple(jnp.zeros((16,), jnp.float32) for _ in range(2 * _CB))
    accs = lax.fori_loop(0, _NCHUNK, chunk_body, accs)
    for cb in range(_CB):
        statv[0, pl.ds(cb * 16, 16)] = accs[cb]
        statv[1, pl.ds(cb * 16, 16)] = accs[_CB + cb]
    pltpu.sync_copy(statv, stats_hbm.at[wid])


@functools.partial(
    pl.kernel,
    out_type=jax.ShapeDtypeStruct((_NC, _NP, _H), jnp.float32),
    mesh=_mesh,
    scratch_types=[
        pltpu.VMEM((_CHUNK,), jnp.int32),
        pltpu.VMEM((_CHUNK, _H), jnp.float32),
        pltpu.VMEM((_CHUNK, _H), jnp.float32),
        pltpu.VMEM((_NW, 2, _H), jnp.float32),
        pltpu.VMEM((4, _H), jnp.float32),
        pltpu.VMEM_SHARED((_NP, _H), jnp.float32),
        pltpu.SemaphoreType.DMA,
        pltpu.SemaphoreType.DMA,
    ],
)
def _sc_phase2(h_hbm, irec_hbm, stats_hbm, params_hbm, zeros_hbm, partial_hbm,
               idx_v, bufh, bufc, statall, parv, accum, sem_h, sem_i):
    cid = lax.axis_index("c")
    sid = lax.axis_index("s")
    base = (sid * _NC + cid) * _EPT

    pltpu.sync_copy(stats_hbm, statall)
    pltpu.sync_copy(params_hbm, parv)

    def stat_body(i, a):
        out = list(a)
        for cb in range(_CB):
            sl = pl.ds(cb * 16, 16)
            out[cb] = out[cb] + statall[i, 0, sl]
            out[_CB + cb] = out[_CB + cb] + statall[i, 1, sl]
        return tuple(out)

    sums = lax.fori_loop(
        0, _NW, stat_body, tuple(jnp.zeros((16,), jnp.float32) for _ in range(2 * _CB))
    )

    inv_e = jnp.float32(1.0 / _E)
    a_vecs = []
    c_vecs = []
    w_vecs = []
    for cb in range(_CB):
        sl = pl.ds(cb * 16, 16)
        mean = sums[cb] * inv_e
        var = sums[_CB + cb] * inv_e - mean * mean
        x = var + jnp.float32(1e-5)
        xi = lax.bitcast_convert_type(x, jnp.int32)
        yi = jnp.full((16,), 0x5F3759DF, jnp.int32) - lax.shift_right_logical(
            xi, jnp.full((16,), 1, jnp.int32)
        )
        y = lax.bitcast_convert_type(yi, jnp.float32)
        half = jnp.float32(0.5) * x
        for _ in range(4):
            y = y * (jnp.float32(1.5) - half * y * y)
        av = parv[0, sl] * y
        a_vecs.append(av)
        c_vecs.append(parv[1, sl] - mean * av)
        w_vecs.append(parv[2, sl])
    bev = parv[3, pl.ds(0, 16)]
    lane = lax.iota(jnp.int32, 16)
    perms = [lane ^ jnp.full((16,), sh, jnp.int32) for sh in (8, 4, 2, 1)]
    _gdn = lax.GatherDimensionNumbers(
        offset_dims=(), collapsed_slice_dims=(0,), start_index_map=(0,)
    )

    def _lane_shuffle(x, perm):
        return lax.gather(
            x,
            perm[:, None],
            dimension_numbers=_gdn,
            slice_sizes=(1,),
            mode=lax.GatherScatterMode.PROMISE_IN_BOUNDS,
        )

    row0 = sid * _RPT
    pltpu.sync_copy(zeros_hbm.at[pl.ds(row0, _RPT)], accum.at[pl.ds(row0, _RPT)])
    plsc.subcore_barrier()

    def chunk_body(it, _):
        off = base + it * _CHUNK
        ch = pltpu.async_copy(h_hbm.at[pl.ds(off, _CHUNK)], bufh, sem_h)
        ci = pltpu.async_copy(irec_hbm.at[pl.ds(off, _CHUNK)], idx_v, sem_i)
        ch.wait()
        ci.wait()

        def edge_body(e, __):
            svs = []
            dot = jnp.zeros((16,), jnp.float32)
            for cb in range(_CB):
                sl = pl.ds(cb * 16, 16)
                m = bufh[e, sl] * a_vecs[cb] + c_vecs[cb]
                sg = jnp.float32(1.0) / (jnp.float32(1.0) + jnp.exp(-m))
                sv = m * sg
                svs.append(sv)
                dot = dot + sv * w_vecs[cb]
            for perm in perms:
                dot = dot + _lane_shuffle(dot, perm)
            wv = dot + bev
            w = jnp.float32(1.0) / (jnp.float32(1.0) + jnp.exp(-wv))
            for cb in range(_CB):
                bufc[e, pl.ds(cb * 16, 16)] = svs[cb] * w
            return 0

        lax.fori_loop(0, _CHUNK, edge_body, 0)
        pltpu.sync_copy(bufc, accum.at[idx_v], add=True)
        return 0

    lax.fori_loop(0, _NCHUNK, chunk_body, 0)
    plsc.subcore_barrier()
    pltpu.sync_copy(
        accum.at[pl.ds(row0, _RPT)], partial_hbm.at[cid, pl.ds(row0, _RPT)]
    )


def kernel(x_send, x_rec, index_send, index_rec, edge_attr,
           W1, b1, bn_gamma, bn_beta, We, be):
    wa = W1[:_H]
    wb = W1[_H:2 * _H]
    wc = W1[2 * _H:]

    p, q = pl.pallas_call(
        _pq_body,
        out_shape=(
            jax.ShapeDtypeStruct((_N, _H), jnp.float32),
            jax.ShapeDtypeStruct((_N, _H), jnp.float32),
        ),
    )(x_send, x_rec, wa, wb)

    rb = pl.pallas_call(
        _r_body,
        grid=(_E // _RBLK,),
        in_specs=[
            pl.BlockSpec((_RBLK, _I), lambda i: (i, 0)),
            pl.BlockSpec((_I, _H), lambda i: (0, 0)),
            pl.BlockSpec((_H,), lambda i: (0,)),
        ],
        out_specs=pl.BlockSpec((_RBLK, _H), lambda i: (i, 0)),
        out_shape=jax.ShapeDtypeStruct((_E, _H), jnp.float32),
    )(edge_attr, wc, b1)

    h, stats = _sc_phase1(p, q, rb, index_send, index_rec)

    params = jnp.stack(
        [bn_gamma, bn_beta, We.reshape(_H), jnp.broadcast_to(be, (_H,))]
    )
    zeros = jnp.zeros((_NP, _H), jnp.float32)
    partial = _sc_phase2(h, index_rec, stats, params, zeros)

    out = pl.pallas_call(
        _add_body,
        out_shape=jax.ShapeDtypeStruct((_N, _H), jnp.float32),
    )(partial[0, :_N], partial[1, :_N])
    return out


# speedup vs baseline: 4.3206x; 1.6418x over previous
import functools

import jax
import jax.numpy as jnp
from jax import lax
from jax.experimental import pallas as pl
from jax.experimental.pallas import tpu as pltpu
from jax.experimental.pallas import tpu_sc as plsc

_N = 10000
_E = 320000
_H = 128
_I = 16

_NC = 2
_NS = 16
_NW = _NC * _NS
_EPT = _E // _NW
_CHUNK = 80
_NCHUNK = _EPT // _CHUNK
_CB = _H // 16
_NP = 10112
_RPT = _NP // _NS

_mesh = plsc.VectorSubcoreMesh(
    core_axis_name="c", subcore_axis_name="s", num_cores=_NC, num_subcores=_NS
)


def _pq_body(xs_ref, xr_ref, wa_ref, wb_ref, p_ref, q_ref):
    p_ref[...] = jnp.dot(xs_ref[...], wa_ref[...], preferred_element_type=jnp.float32)
    q_ref[...] = jnp.dot(xr_ref[...], wb_ref[...], preferred_element_type=jnp.float32)


_RBLK = 8000


def _r_body(ea_ref, wc_ref, b1_ref, r_ref):
    r_ref[...] = (
        jnp.dot(ea_ref[...], wc_ref[...], preferred_element_type=jnp.float32)
        + b1_ref[...]
    )


def _add_body(a_ref, b_ref, o_ref):
    o_ref[...] = a_ref[...] + b_ref[...]


def _fold_body(stats_ref, g_ref, b_ref, we_ref, be_ref, par_ref):
    sums = jnp.sum(stats_ref[...], axis=0)
    inv_e = jnp.float32(1.0 / _E)
    mean = sums[0] * inv_e
    var = sums[1] * inv_e - mean * mean
    a = g_ref[...] * lax.rsqrt(var + jnp.float32(1e-5))
    par_ref[0, :] = a
    par_ref[1, :] = b_ref[...] - mean * a
    par_ref[2, :] = we_ref[...]
    par_ref[3, :] = be_ref[...]


@functools.partial(
    pl.kernel,
    out_type=(
        jax.ShapeDtypeStruct((_E, _H), jnp.float32),
        jax.ShapeDtypeStruct((_NW, 2, _H), jnp.float32),
    ),
    mesh=_mesh,
    scratch_types=(
        [pltpu.VMEM((_CHUNK,), jnp.int32)] * 8
        + [pltpu.VMEM((_CHUNK, _H), jnp.float32)] * 6
        + [
            pltpu.VMEM((2, _H), jnp.float32),
            pltpu.SemaphoreType.DMA,
            pltpu.SemaphoreType.DMA,
            pltpu.SemaphoreType.DMA,
            pltpu.SemaphoreType.DMA,
            pltpu.SemaphoreType.DMA,
            pltpu.SemaphoreType.DMA,
        ]
    ),
)
def _sc_phase1(p_hbm, q_hbm, rb_hbm, isend_hbm, irec_hbm, h_hbm, stats_hbm,
               ixs0, ixs1, ixs2, ixs3, ixr0, ixr1, ixr2, ixr3,
               bp0, bq0, br0, bp1, bq1, br1, statv,
               sg0, sg1, sh0, sh1, si0, si1):
    wid = lax.axis_index("s") * _NC + lax.axis_index("c")
    base = wid * _EPT

    ixs = (ixs0, ixs1, ixs2, ixs3)
    ixr = (ixr0, ixr1, ixr2, ixr3)
    bps = (bp0, bp1)
    bqs = (bq0, bq1)
    brs = (br0, br1)
    sgs = (sg0, sg1)
    shs = (sh0, sh1)
    sis = (si0, si1)

    def i_descs(it, r4):
        off = base + it * _CHUNK
        si = sis[r4 % 2]
        return (
            pltpu.make_async_copy(isend_hbm.at[pl.ds(off, _CHUNK)], ixs[r4], si),
            pltpu.make_async_copy(irec_hbm.at[pl.ds(off, _CHUNK)], ixr[r4], si),
        )

    def g_descs(it, r4):
        p2 = r4 % 2
        off = it * _CHUNK
        return (
            pltpu.make_async_copy(p_hbm.at[ixs[r4]], bps[p2], sgs[p2]),
            pltpu.make_async_copy(q_hbm.at[ixr[r4]], bqs[p2], sgs[p2]),
            pltpu.make_async_copy(
                rb_hbm.at[pl.ds(base + off, _CHUNK)], brs[p2], sgs[p2]
            ),
        )

    def h_desc(it, p2):
        return pltpu.make_async_copy(
            brs[p2], h_hbm.at[pl.ds(base + it * _CHUNK, _CHUNK)], shs[p2]
        )

    def compute(it, p2, accs):
        bp, bq, br = bps[p2], bqs[p2], brs[p2]

        def edge_body(e, a):
            out = list(a)
            for cb in range(_CB):
                sl = pl.ds(cb * 16, 16)
                t = bp[e, sl] + bq[e, sl] + br[e, sl]
                br[e, sl] = t
                out[cb] = out[cb] + t
                out[_CB + cb] = out[_CB + cb] + t * t
            return tuple(out)

        accs = lax.fori_loop(0, _CHUNK, edge_body, accs)
        h_desc(it, p2).start()
        return accs

    def step(it, k, accs):
        @pl.when(jnp.asarray(it + 1 < _NCHUNK))
        def _():
            for c in i_descs(it + 1, (k + 1) % 4):
                c.wait()

            @pl.when(jnp.asarray(it >= 1))
            def __():
                h_desc(it - 1, (k + 1) % 2).wait()

            for c in g_descs(it + 1, (k + 1) % 4):
                c.start()

        for c in g_descs(it, k):
            c.wait()
        accs = compute(it, k % 2, accs)

        @pl.when(jnp.asarray(it + 2 < _NCHUNK))
        def _():
            for c in i_descs(it + 2, (k + 2) % 4):
                c.start()

        return accs

    for c in i_descs(0, 0):
        c.start()
    for c in i_descs(1, 1):
        c.start()
    for c in i_descs(0, 0):
        c.wait()
    for c in g_descs(0, 0):
        c.start()

    def quad_body(g, accs):
        it0 = 4 * g
        for k in range(4):
            accs = step(it0 + k, k, accs)
        return accs

    accs = tuple(jnp.zeros((16,), jnp.float32) for _ in range(2 * _CB))
    accs = lax.fori_loop(0, _NCHUNK // 4, quad_body, accs)
    accs = step(_NCHUNK - 1, 0, accs)
    h_desc(_NCHUNK - 2, 1).wait()
    h_desc(_NCHUNK - 1, 0).wait()

    for cb in range(_CB):
        statv[0, pl.ds(cb * 16, 16)] = accs[cb]
        statv[1, pl.ds(cb * 16, 16)] = accs[_CB + cb]
    pltpu.sync_copy(statv, stats_hbm.at[wid])


@functools.partial(
    pl.kernel,
    out_type=jax.ShapeDtypeStruct((_NC, _NP, _H), jnp.float32),
    mesh=_mesh,
    scratch_types=[
        pltpu.VMEM((_CHUNK,), jnp.int32),
        pltpu.VMEM((_CHUNK,), jnp.int32),
        pltpu.VMEM((_CHUNK,), jnp.int32),
        pltpu.VMEM((_CHUNK,), jnp.int32),
        pltpu.VMEM((_CHUNK, _H), jnp.float32),
        pltpu.VMEM((_CHUNK, _H), jnp.float32),
        pltpu.VMEM((_CHUNK, _H), jnp.float32),
        pltpu.VMEM((_CHUNK, _H), jnp.float32),
        pltpu.VMEM((4, _H), jnp.float32),
        pltpu.VMEM_SHARED((_NP, _H), jnp.float32),
        pltpu.SemaphoreType.DMA,
        pltpu.SemaphoreType.DMA,
        pltpu.SemaphoreType.DMA,
        pltpu.SemaphoreType.DMA,
    ],
)
def _sc_phase2(h_hbm, irec_hbm, params_hbm, zeros_hbm, partial_hbm,
               ix0, ix1, ix2, ix3, bh0, bh1, bc0, bc1, parv, accum,
               sl0, sl1, ss0, ss1):
    cid = lax.axis_index("c")
    sid = lax.axis_index("s")
    base = (sid * _NC + cid) * _EPT

    pltpu.sync_copy(params_hbm, parv)

    a_vecs = []
    c_vecs = []
    w_vecs = []
    for cb in range(_CB):
        sl = pl.ds(cb * 16, 16)
        a_vecs.append(parv[0, sl])
        c_vecs.append(parv[1, sl])
        w_vecs.append(parv[2, sl])
    bev = parv[3, pl.ds(0, 16)]
    lane = lax.iota(jnp.int32, 16)
    perms = [lane ^ jnp.full((16,), sh, jnp.int32) for sh in (8, 4, 2, 1)]
    _gdn = lax.GatherDimensionNumbers(
        offset_dims=(), collapsed_slice_dims=(0,), start_index_map=(0,)
    )

    def _lane_shuffle(x, perm):
        return lax.gather(
            x,
            perm[:, None],
            dimension_numbers=_gdn,
            slice_sizes=(1,),
            mode=lax.GatherScatterMode.PROMISE_IN_BOUNDS,
        )

    ixs = (ix0, ix1, ix2, ix3)
    bhs = (bh0, bh1)
    bcs = (bc0, bc1)
    sls = (sl0, sl1)
    sss = (ss0, ss1)

    def l_descs(it, p2, p4):
        off = base + it * _CHUNK
        return (
            pltpu.make_async_copy(h_hbm.at[pl.ds(off, _CHUNK)], bhs[p2], sls[p2]),
            pltpu.make_async_copy(irec_hbm.at[pl.ds(off, _CHUNK)], ixs[p4], sls[p2]),
        )

    def sc_desc(p2, p4):
        return pltpu.make_async_copy(bcs[p2], accum.at[ixs[p4]], sss[p2])

    def issue_loads(it, p2, p4):
        for c in l_descs(it, p2, p4):
            c.start()

    def compute(it, p2, p4):
        bh = bhs[p2]
        bc = bcs[p2]

        @pl.when(jnp.asarray(it >= 2))
        def _():
            sc_desc(p2, (p4 + 2) % 4).wait()

        def edge_body(e, __):
            svs = []
            dot = jnp.zeros((16,), jnp.float32)
            for cb in range(_CB):
                sl = pl.ds(cb * 16, 16)
                m = bh[e, sl] * a_vecs[cb] + c_vecs[cb]
                sg = jnp.float32(1.0) / (jnp.float32(1.0) + jnp.exp(-m))
                sv = m * sg
                svs.append(sv)
                dot = dot + sv * w_vecs[cb]
            for perm in perms:
                dot = dot + _lane_shuffle(dot, perm)
            wv = dot + bev
            w = jnp.float32(1.0) / (jnp.float32(1.0) + jnp.exp(-wv))
            for cb in range(_CB):
                bc[e, pl.ds(cb * 16, 16)] = svs[cb] * w
            return 0

        lax.fori_loop(0, _CHUNK, edge_body, 0)
        pltpu.async_copy(bcs[p2], accum.at[ixs[p4]], sss[p2], add=True)

    issue_loads(0, 0, 0)
    issue_loads(1, 1, 1)

    row0 = sid * _RPT
    pltpu.sync_copy(zeros_hbm.at[pl.ds(row0, _RPT)], accum.at[pl.ds(row0, _RPT)])
    plsc.subcore_barrier()

    def quad_body(g, _):
        it0 = 4 * g
        for k in range(4):
            it = it0 + k
            p2 = k % 2
            for c in l_descs(it, p2, k):
                c.wait()
            compute(it, p2, k)

            @pl.when(it + 2 < _NCHUNK)
            def _():
                issue_loads(it + 2, p2, (k + 2) % 4)

        return 0

    lax.fori_loop(0, _NCHUNK // 4, quad_body, 0)
    itl = _NCHUNK - 1
    for c in l_descs(itl, 0, 0):
        c.wait()
    compute(itl, 0, 0)
    sc_desc(1, 3).wait()
    sc_desc(0, 0).wait()

    plsc.subcore_barrier()
    pltpu.sync_copy(
        accum.at[pl.ds(row0, _RPT)], partial_hbm.at[cid, pl.ds(row0, _RPT)]
    )


def kernel(x_send, x_rec, index_send, index_rec, edge_attr,
           W1, b1, bn_gamma, bn_beta, We, be):
    wa = W1[:_H]
    wb = W1[_H:2 * _H]
    wc = W1[2 * _H:]

    p, q = pl.pallas_call(
        _pq_body,
        out_shape=(
            jax.ShapeDtypeStruct((_N, _H), jnp.float32),
            jax.ShapeDtypeStruct((_N, _H), jnp.float32),
        ),
    )(x_send, x_rec, wa, wb)

    rb = pl.pallas_call(
        _r_body,
        grid=(_E // _RBLK,),
        in_specs=[
            pl.BlockSpec((_RBLK, _I), lambda i: (i, 0)),
            pl.BlockSpec((_I, _H), lambda i: (0, 0)),
            pl.BlockSpec((_H,), lambda i: (0,)),
        ],
        out_specs=pl.BlockSpec((_RBLK, _H), lambda i: (i, 0)),
        out_shape=jax.ShapeDtypeStruct((_E, _H), jnp.float32),
    )(edge_attr, wc, b1)

    h, stats = _sc_phase1(p, q, rb, index_send, index_rec)

    params = pl.pallas_call(
        _fold_body,
        out_shape=jax.ShapeDtypeStruct((4, _H), jnp.float32),
    )(stats, bn_gamma, bn_beta, We.reshape(_H), jnp.broadcast_to(be, (_H,)))
    zeros = jnp.zeros((_NP, _H), jnp.float32)
    partial = _sc_phase2(h, index_rec, params, zeros)

    out = pl.pallas_call(
        _add_body,
        out_shape=jax.ShapeDtypeStruct((_N, _H), jnp.float32),
    )(partial[0, :_N], partial[1, :_N])
    return out
